# speedup vs baseline: 1.5536x; 1.5536x over previous
import functools

import jax
import jax.numpy as jnp
from jax import lax
from jax.experimental import pallas as pl
from jax.experimental.pallas import tpu as pltpu
from jax.experimental.pallas import tpu_sc as plsc

HIDDEN = 1024
B_TOTAL = 4 * 4096

_info = plsc.get_sparse_core_info()
NC, NS = _info.num_cores, _info.num_subcores
NW = NC * NS
B_PER_W = B_TOTAL // NW
CHUNK = 64
N_CHUNKS = B_PER_W // CHUNK

_mesh = plsc.VectorSubcoreMesh(core_axis_name="c", subcore_axis_name="s")


@functools.partial(
    pl.kernel,
    mesh=_mesh,
    out_type=jax.ShapeDtypeStruct((B_TOTAL, HIDDEN), jnp.float32),
    scratch_types=[
        pltpu.VMEM((B_PER_W,), jnp.int32),
        pltpu.VMEM((CHUNK, HIDDEN), jnp.float32),
        pltpu.SemaphoreType.DMA,
    ],
)
def _gather_kernel(ids_hbm, table_hbm, out_hbm, idx_v, rows_v, sem):
    wid = lax.axis_index("s") * NC + lax.axis_index("c")
    base = wid * B_PER_W
    pltpu.sync_copy(ids_hbm.at[pl.ds(base, B_PER_W)], idx_v)
    for j in range(N_CHUNKS):
        idx_chunk = idx_v.at[pl.ds(j * CHUNK, CHUNK)]
        pltpu.async_copy(table_hbm.at[idx_chunk], rows_v, sem).wait()
        pltpu.sync_copy(rows_v, out_hbm.at[pl.ds(base + j * CHUNK, CHUNK)])


def kernel(input_ids, embed_tokens):
    flat = input_ids.reshape(-1).astype(jnp.int32)
    out = _gather_kernel(flat, embed_tokens)
    return out.reshape(input_ids.shape + (HIDDEN,))


# speedup vs baseline: 1.6726x; 1.0766x over previous
import functools

import jax
import jax.numpy as jnp
from jax import lax
from jax.experimental import pallas as pl
from jax.experimental.pallas import tpu as pltpu
from jax.experimental.pallas import tpu_sc as plsc

HIDDEN = 1024
B_TOTAL = 4 * 4096

_info = plsc.get_sparse_core_info()
NC, NS = _info.num_cores, _info.num_subcores
NW = NC * NS
B_PER_W = B_TOTAL // NW
CHUNK = 32
N_CHUNKS = B_PER_W // CHUNK
NBUF = 3


_mesh = plsc.VectorSubcoreMesh(core_axis_name="c", subcore_axis_name="s")


@functools.partial(
    pl.kernel,
    mesh=_mesh,
    out_type=jax.ShapeDtypeStruct((B_TOTAL, HIDDEN), jnp.float32),
    scratch_types=[
        pltpu.VMEM((B_PER_W,), jnp.int32),
        [pltpu.VMEM((CHUNK, HIDDEN), jnp.float32) for _ in range(NBUF)],
        [pltpu.SemaphoreType.DMA for _ in range(NBUF)],
        [pltpu.SemaphoreType.DMA for _ in range(NBUF)],
    ],
)
def _gather_kernel(ids_hbm, table_hbm, out_hbm, idx_v, rows, gsem, ssem):
    wid = lax.axis_index("s") * NC + lax.axis_index("c")
    base = wid * B_PER_W
    pltpu.sync_copy(ids_hbm.at[pl.ds(base, B_PER_W)], idx_v)

    def start_gather(j):
        b = j % NBUF
        idx_chunk = idx_v.at[pl.ds(j * CHUNK, CHUNK)]
        return pltpu.async_copy(table_hbm.at[idx_chunk], rows[b], gsem[b])

    def start_scatter(j):
        b = j % NBUF
        return pltpu.async_copy(
            rows[b], out_hbm.at[pl.ds(base + j * CHUNK, CHUNK)], ssem[b])

    gathers = {}
    scatters = {}
    for j in range(min(NBUF, N_CHUNKS)):
        gathers[j] = start_gather(j)
    for j in range(N_CHUNKS):
        gathers[j].wait()
        scatters[j] = start_scatter(j)
        nxt = j + NBUF
        if nxt < N_CHUNKS:
            scatters[nxt - NBUF].wait()
            gathers[nxt] = start_gather(nxt)
    for j in range(max(0, N_CHUNKS - NBUF), N_CHUNKS):
        scatters[j].wait()


def kernel(input_ids, embed_tokens):
    flat = input_ids.reshape(-1).astype(jnp.int32)
    out = _gather_kernel(flat, embed_tokens)
    return out.reshape(input_ids.shape + (HIDDEN,))


# speedup vs baseline: 1.6800x; 1.0044x over previous
import functools

import jax
import jax.numpy as jnp
from jax import lax
from jax.experimental import pallas as pl
from jax.experimental.pallas import tpu as pltpu
from jax.experimental.pallas import tpu_sc as plsc

HIDDEN = 1024
BATCH = 4
SEQ = 4096
B_TOTAL = BATCH * SEQ

_info = plsc.get_sparse_core_info()
NC, NS = _info.num_cores, _info.num_subcores
NW = NC * NS
B_PER_W = B_TOTAL // NW
CHUNK = 32
N_CHUNKS = B_PER_W // CHUNK
NBUF = 3


_mesh = plsc.VectorSubcoreMesh(core_axis_name="c", subcore_axis_name="s")


@functools.partial(
    pl.kernel,
    mesh=_mesh,
    out_type=jax.ShapeDtypeStruct((BATCH, SEQ, HIDDEN), jnp.float32),
    scratch_types=[
        pltpu.VMEM((B_PER_W,), jnp.int32),
        [pltpu.VMEM((CHUNK, HIDDEN), jnp.float32) for _ in range(NBUF)],
        [pltpu.SemaphoreType.DMA for _ in range(NBUF)],
        [pltpu.SemaphoreType.DMA for _ in range(NBUF)],
    ],
)
def _gather_kernel(ids_hbm, table_hbm, out_hbm, idx_v, rows, gsem, ssem):
    wid = lax.axis_index("s") * NC + lax.axis_index("c")
    bi = wid // (NW // BATCH)
    off = (wid % (NW // BATCH)) * B_PER_W
    pltpu.sync_copy(ids_hbm.at[bi, pl.ds(off, B_PER_W)], idx_v)

    def start_gather(j):
        b = j % NBUF
        idx_chunk = idx_v.at[pl.ds(j * CHUNK, CHUNK)]
        return pltpu.async_copy(table_hbm.at[idx_chunk], rows[b], gsem[b])

    def start_scatter(j):
        b = j % NBUF
        return pltpu.async_copy(
            rows[b], out_hbm.at[bi, pl.ds(off + j * CHUNK, CHUNK)], ssem[b])

    gathers = {}
    scatters = {}
    for j in range(min(NBUF, N_CHUNKS)):
        gathers[j] = start_gather(j)
    for j in range(N_CHUNKS):
        gathers[j].wait()
        scatters[j] = start_scatter(j)
        nxt = j + NBUF
        if nxt < N_CHUNKS:
            scatters[nxt - NBUF].wait()
            gathers[nxt] = start_gather(nxt)
    for j in range(max(0, N_CHUNKS - NBUF), N_CHUNKS):
        scatters[j].wait()


def kernel(input_ids, embed_tokens):
    return _gather_kernel(input_ids.astype(jnp.int32), embed_tokens)
